# baseline (speedup 1.0000x reference)
import jax
import jax.numpy as jnp
import numpy as np
from jax.experimental import pallas as pl
from jax.experimental.pallas import tpu as pltpu

_B, _CIN, _H, _W = 4, 3, 512, 512
_S = 16
_CO = 128
_FH, _FW = _H // _S, _W // _S
_K = _CIN * _S * _S
_HB = 8


def _patch_conv_kernel(x_ref, e_ref, w_ref, b_ref, o_ref):
    xb = x_ref[...].reshape(_CIN * _HB * _S, _W)
    z = jnp.dot(xb, e_ref[...], preferred_element_type=jnp.float32)
    z4 = z.reshape(_CIN, _HB, _S, _W)
    v = jnp.stack([z4[:, :, :, kw * _FW:(kw + 1) * _FW] for kw in range(_S)])
    w = w_ref[...]
    b = b_ref[...]
    for h in range(_HB):
        zh = v[:, :, h].reshape(_K, _FW)
        acc = jnp.dot(w, zh, preferred_element_type=jnp.float32)
        o_ref[0, :, h, :] = jnp.maximum(acc + b, 0.0)


def kernel(x, gts, Wc, bc):
    del gts
    col = np.arange(_W)
    dst = (col % _S) * _FW + col // _S
    em = jnp.asarray((dst[:, None] == np.arange(_W)[None, :]),
                     dtype=jnp.float32)
    wm = jnp.transpose(Wc, (0, 3, 1, 2)).reshape(_CO, _K)
    bm = bc.reshape(_CO, 1)
    xs = x.reshape(_B, _CIN, _FH // _HB, _HB * _S, _W)
    out = pl.pallas_call(
        _patch_conv_kernel,
        grid=(_B, _FH // _HB),
        in_specs=[
            pl.BlockSpec((1, _CIN, 1, _HB * _S, _W),
                         lambda b, h: (b, 0, h, 0, 0)),
            pl.BlockSpec((_W, _W), lambda b, h: (0, 0)),
            pl.BlockSpec((_CO, _K), lambda b, h: (0, 0)),
            pl.BlockSpec((_CO, 1), lambda b, h: (0, 0)),
        ],
        out_specs=pl.BlockSpec((1, _CO, _HB, _FW), lambda b, h: (b, 0, h, 0)),
        out_shape=jax.ShapeDtypeStruct((_B, _CO, _FH, _FW), jnp.float32),
        compiler_params=pltpu.CompilerParams(
            dimension_semantics=("parallel", "parallel")),
    )(xs, em, wm, bm)
    return out


# speedup vs baseline: 1.0890x; 1.0890x over previous
import jax
import jax.numpy as jnp
from jax.experimental import pallas as pl
from jax.experimental.pallas import tpu as pltpu

_B, _CIN, _H, _W = 4, 3, 512, 512
_S = 16
_CO = 128
_FH, _FW = _H // _S, _W // _S
_K = _CIN * _S * _S
_HB = 16


def _patch_conv_kernel(x_ref, w_ref, b_ref, o_ref, e_ref):
    @pl.when(jnp.logical_and(pl.program_id(0) == 0, pl.program_id(1) == 0))
    def _init_perm():
        src = jax.lax.broadcasted_iota(jnp.int32, (_W, _W), 0)
        dstc = jax.lax.broadcasted_iota(jnp.int32, (_W, _W), 1)
        dst = (src % _S) * _FW + src // _S
        e_ref[...] = (dst == dstc).astype(jnp.float32)

    xb = x_ref[...].reshape(_CIN * _HB * _S, _W)
    z = jnp.dot(xb, e_ref[...], preferred_element_type=jnp.float32)
    z4 = z.reshape(_CIN, _HB, _S, _W)
    v = jnp.stack([z4[:, :, :, kw * _FW:(kw + 1) * _FW] for kw in range(_S)])
    w = w_ref[...]
    b = b_ref[...]
    for h in range(_HB):
        zh = v[:, :, h].reshape(_K, _FW)
        acc = jnp.dot(w, zh, preferred_element_type=jnp.float32)
        o_ref[0, :, h, :] = jnp.maximum(acc + b, 0.0)


def kernel(x, gts, Wc, bc):
    del gts
    wm = jnp.transpose(Wc, (0, 3, 1, 2)).reshape(_CO, _K)
    bm = bc.reshape(_CO, 1)
    xs = x.reshape(_B, _CIN, _FH // _HB, _HB * _S, _W)
    out = pl.pallas_call(
        _patch_conv_kernel,
        grid=(_B, _FH // _HB),
        in_specs=[
            pl.BlockSpec((1, _CIN, 1, _HB * _S, _W),
                         lambda b, h: (b, 0, h, 0, 0)),
            pl.BlockSpec((_CO, _K), lambda b, h: (0, 0)),
            pl.BlockSpec((_CO, 1), lambda b, h: (0, 0)),
        ],
        scratch_shapes=[pltpu.VMEM((_W, _W), jnp.float32)],
        out_specs=pl.BlockSpec((1, _CO, _HB, _FW), lambda b, h: (b, 0, h, 0)),
        out_shape=jax.ShapeDtypeStruct((_B, _CO, _FH, _FW), jnp.float32),
        compiler_params=pltpu.CompilerParams(
            dimension_semantics=("arbitrary", "arbitrary")),
    )(xs, wm, bm)
    return out
